# baseline (speedup 1.0000x reference)
import functools

import jax
import jax.numpy as jnp
from jax import lax
from jax.experimental import pallas as pl
from jax.experimental.pallas import tpu as pltpu
from jax.experimental.pallas import tpu_sc as plsc

D = 64
CH = 128
BLK = 16
EPB = CH * BLK
NSUB = 16
NCORE = 2


def _partition_sc(src2, map2, half):
    R = src2.shape[0]
    rps = R // NSUB
    CB = -(-(rps * CH + CH) // EPB) * BLK
    mesh = plsc.VectorSubcoreMesh(core_axis_name="c", subcore_axis_name="s")

    @functools.partial(
        pl.kernel,
        mesh=mesh,
        compiler_params=pltpu.CompilerParams(use_tc_tiling_on_sc=False,
                                             needs_layout_passes=False),
        out_type=(
            jax.ShapeDtypeStruct((NCORE, NSUB, CB * CH), jnp.int32),
            jax.ShapeDtypeStruct((NCORE, NSUB, CB * CH), jnp.int32),
            jax.ShapeDtypeStruct((NCORE, NSUB, 16), jnp.int32),
        ),
        scratch_types=[
            pltpu.VMEM((BLK, CH), jnp.int32),
            pltpu.VMEM((BLK, CH), jnp.int32),
            pltpu.VMEM((CB * CH + 16,), jnp.int32),
            pltpu.VMEM((CB * CH + 16,), jnp.int32),
            pltpu.VMEM((16,), jnp.int32),
        ],
    )
    def part_kernel(src_hbm, map_hbm, csrc_hbm, cmap_hbm, cnt_hbm,
                    in_s, in_m, out_s, out_m, cnt_v):
        c = lax.axis_index("c")
        s = lax.axis_index("s")
        base = s * rps
        nblk = rps // BLK

        def blk_body(blk, cnt):
            pltpu.sync_copy(src_hbm.at[pl.ds(base + blk * BLK, BLK)], in_s)
            pltpu.sync_copy(map_hbm.at[c].at[pl.ds(base + blk * BLK, BLK)],
                            in_m)

            lane = lax.iota(jnp.int32, 16)

            def vec_body(i, cnt):
                j = i // (CH // 16)
                k = i % (CH // 16)
                mv = in_m[j, pl.ds(k * 16, 16)]
                sv = in_s[j, pl.ds(k * 16, 16)]
                keep = mv < half
                pfx = jnp.cumsum(jnp.where(keep, 1, 0).astype(jnp.int32))
                pos = jnp.where(keep, cnt + pfx - 1, CB * CH + lane)
                plsc.store_scatter(out_s, [pos], sv)
                plsc.store_scatter(out_m, [pos], mv)
                return cnt + jnp.max(pfx)
            return lax.fori_loop(0, BLK * (CH // 16), vec_body, cnt)
        cnt = lax.fori_loop(0, nblk, blk_body, jnp.int32(0))

        iota = lax.iota(jnp.int32, 16)
        dummy_m = half + ((iota * 5 + s) & 63)
        for k in range(CH // 16):
            dummy_s = ((s * 16 + k) * 499 + iota * 31) & 16383
            out_s[pl.ds(cnt + k * 16, 16)] = dummy_s
            out_m[pl.ds(cnt + k * 16, 16)] = dummy_m
        cnt_pad = ((cnt + CH - 1) // CH) * CH

        cnt_v[...] = jnp.broadcast_to(cnt_pad, (16,))
        pltpu.sync_copy(out_s.at[pl.ds(0, CB * CH)], csrc_hbm.at[c].at[s])
        pltpu.sync_copy(out_m.at[pl.ds(0, CB * CH)], cmap_hbm.at[c].at[s])
        pltpu.sync_copy(cnt_v, cnt_hbm.at[c].at[s])

    return part_kernel(src2, map2)


def _segsum_sc(f, csrc4, cmap4, counts, half, acc_rows):
    CB = csrc4.shape[2]
    rows_per_tile = acc_rows // NSUB
    ZR = 56
    nzcopies = rows_per_tile // ZR
    NBUF = 2

    mesh = plsc.VectorSubcoreMesh(core_axis_name="c", subcore_axis_name="s")

    @functools.partial(
        pl.kernel,
        mesh=mesh,
        compiler_params=pltpu.CompilerParams(use_tc_tiling_on_sc=False,
                                             needs_layout_passes=False),
        out_type=jax.ShapeDtypeStruct((NCORE, acc_rows, D), jnp.float32),
        scratch_types=[
            pltpu.VMEM((BLK, CH), jnp.int32),
            pltpu.VMEM((BLK, CH), jnp.int32),
            pltpu.VMEM((NBUF, CH, D), jnp.float32),
            pltpu.VMEM((ZR, D), jnp.float32),
            pltpu.VMEM((16,), jnp.int32),
            pltpu.VMEM_SHARED((acc_rows, D), jnp.float32),
            pltpu.SemaphoreType.DMA,
        ],
    )
    def seg_kernel(f_hbm, src_hbm, map_hbm, cnt_hbm, out_hbm,
                   src_v, map_v, rows_v, zero_v, cnt_v, acc_sh, gsem):
        c = lax.axis_index("c")
        s = lax.axis_index("s")

        def zfill(i, _):
            def zfill2(k, _):
                zero_v[i, pl.ds(k * 16, 16)] = jnp.zeros((16,), jnp.float32)
                return 0
            return lax.fori_loop(0, D // 16, zfill2, 0)
        lax.fori_loop(0, ZR, zfill, 0)

        def zcopy(t, _):
            pltpu.sync_copy(zero_v,
                            acc_sh.at[pl.ds(s * rows_per_tile + t * ZR, ZR)])
            return 0
        lax.fori_loop(0, nzcopies, zcopy, 0)
        pltpu.sync_copy(cnt_hbm.at[c].at[s], cnt_v)
        plsc.subcore_barrier()

        nch = jnp.max(cnt_v[...]) // CH
        nblk_full = nch // BLK
        rem = nch - nblk_full * BLK

        def load_idx(blk):
            pltpu.sync_copy(src_hbm.at[c].at[s].at[pl.ds(blk * BLK, BLK)],
                            src_v)
            pltpu.sync_copy(map_hbm.at[c].at[s].at[pl.ds(blk * BLK, BLK)],
                            map_v)

        def blk_body(blk, _):
            load_idx(blk)
            cps = [None] * BLK
            cps[0] = pltpu.async_copy(f_hbm.at[src_v.at[0]], rows_v.at[0],
                                      gsem)
            for j in range(BLK):
                cps[j].wait()
                if j + 1 < BLK:
                    cps[j + 1] = pltpu.async_copy(
                        f_hbm.at[src_v.at[j + 1]],
                        rows_v.at[(j + 1) % NBUF], gsem)
                pltpu.sync_copy(rows_v.at[j % NBUF], acc_sh.at[map_v.at[j]],
                                add=True)
            return 0
        lax.fori_loop(0, nblk_full, blk_body, 0)

        @pl.when(rem > 0)
        def _():
            load_idx(nblk_full)

        def rem_body(j, _):
            pltpu.async_copy(f_hbm.at[src_v.at[j]], rows_v.at[0],
                             gsem).wait()
            pltpu.sync_copy(rows_v.at[0], acc_sh.at[map_v.at[j]], add=True)
            return 0
        lax.fori_loop(0, rem, rem_body, 0)

        plsc.subcore_barrier()
        pltpu.sync_copy(acc_sh.at[pl.ds(s * rows_per_tile, rows_per_tile)],
                        out_hbm.at[c].at[pl.ds(s * rows_per_tile,
                                               rows_per_tile)])

    return seg_kernel(f, csrc4, cmap4, counts)


def _dense_layer(f, agg, Wr, Wn, b8, acc):
    NR = f.shape[0]
    BR = 2000

    def body(f_ref, a_ref, wr_ref, wn_ref, b_ref, acc_ref, fout_ref, aout_ref):
        x = f_ref[...]
        y = jnp.dot(x, wr_ref[...], preferred_element_type=jnp.float32,
                    precision=lax.Precision.HIGHEST)
        y = y + jnp.dot(a_ref[...], wn_ref[...],
                        preferred_element_type=jnp.float32,
                        precision=lax.Precision.HIGHEST)
        y = y + b_ref[0:1, :]
        fout_ref[...] = y
        nrm = jnp.sqrt(jnp.sum(y * y, axis=1, keepdims=True))
        aout_ref[...] = acc_ref[...] + y / jnp.maximum(nrm, 1e-12)

    return pl.pallas_call(
        body,
        grid=(NR // BR,),
        in_specs=[
            pl.BlockSpec((BR, D), lambda i: (i, 0)),
            pl.BlockSpec((BR, D), lambda i: (i, 0)),
            pl.BlockSpec((D, D), lambda i: (0, 0)),
            pl.BlockSpec((D, D), lambda i: (0, 0)),
            pl.BlockSpec((8, D), lambda i: (0, 0)),
            pl.BlockSpec((BR, D), lambda i: (i, 0)),
        ],
        out_specs=[
            pl.BlockSpec((BR, D), lambda i: (i, 0)),
            pl.BlockSpec((BR, D), lambda i: (i, 0)),
        ],
        out_shape=[
            jax.ShapeDtypeStruct((NR, D), jnp.float32),
            jax.ShapeDtypeStruct((NR, D), jnp.float32),
        ],
    )(f, agg, Wr, Wn, b8, acc)


def kernel(users_feature, items_feature, edge_index,
           W_root0, W_rel0, b0, W_root1, W_rel1, b1):
    feats = jnp.concatenate([users_feature, items_feature], axis=0)
    n_nodes = feats.shape[0]
    half = n_nodes // 2
    rows_per_tile = -(-(half + 64) // (16 * 56)) * 56
    acc_rows = rows_per_tile * 16

    src = edge_index[0]
    dst = edge_index[1]
    E = src.shape[0]
    per_sub = -(-E // (NSUB * EPB)) * EPB
    E_pad = per_sub * NSUB
    pad = E_pad - E
    src_p = jnp.concatenate([src, jnp.zeros((pad,), jnp.int32)])
    dst_p = jnp.concatenate([dst, jnp.full((pad,), n_nodes, jnp.int32)])
    src2 = src_p.reshape(E_pad // CH, CH)
    lo = jnp.arange(NCORE, dtype=jnp.int32)[:, None] * half
    t = dst_p[None, :] - lo
    dummy = half + (jnp.arange(E_pad, dtype=jnp.int32)[None, :] & 63)
    map2 = jnp.where((t >= 0) & (t < half), t, dummy).reshape(
        NCORE, E_pad // CH, CH)

    b0b = jnp.broadcast_to(b0.reshape(1, D), (8, D))
    b1b = jnp.broadcast_to(b1.reshape(1, D), (8, D))

    csrc, cmap, counts = _partition_sc(src2, map2, half)
    CB = csrc.shape[2] // CH
    csrc4 = csrc.reshape(NCORE, NSUB, CB, CH)
    cmap4 = cmap.reshape(NCORE, NSUB, CB, CH)

    f = feats
    acc = feats
    for (Wr, Wn, bb) in ((W_root0, W_rel0, b0b), (W_root1, W_rel1, b1b)):
        agg2 = _segsum_sc(f, csrc4, cmap4, counts, half, acc_rows)
        agg = jnp.concatenate([agg2[0, :half], agg2[1, :half]], axis=0)
        f, acc = _dense_layer(f, agg, Wr, Wn, bb, acc)
    return acc
